# baseline (speedup 1.0000x reference)
import functools

import jax
import jax.numpy as jnp
from jax import lax
from jax.experimental import pallas as pl
from jax.experimental.pallas import tpu as pltpu
from jax.experimental.pallas import tpu_sc as plsc

N = 4096
E = 65536
H = 128
OUT = 128
NEXP = 8
NGRAPH = 8
ZDIM = 130

NC = 2
NS = 16
NW = NC * NS
EPT = E // NW
CHUNK = 128
NCHUNK = EPT // CHUNK
NBUF1 = 4
NBUF2 = 2
CHUNK2 = 128
NCHUNK2 = EPT // CHUNK2

BR = 512
BC = 512
EB = 8192


def _enc_body(xs_ref, w_ref, b_ref, out_ref):
    out_ref[...] = jnp.maximum(
        jnp.dot(xs_ref[...], w_ref[...], preferred_element_type=jnp.float32)
        + b_ref[...], 0.0)


def _encode(xs, W_enc, b_enc):
    return pl.pallas_call(
        _enc_body,
        out_shape=jax.ShapeDtypeStruct((N, H), jnp.float32),
    )(xs, W_enc, b_enc)


def _deg_sc(dst2d, zeros_acc, ones_rows):
    mesh = plsc.VectorSubcoreMesh(core_axis_name="c", subcore_axis_name="s")

    @functools.partial(
        pl.kernel,
        out_type=jax.ShapeDtypeStruct((NC, N, H), jnp.float32),
        mesh=mesh,
        scratch_types=[
            pltpu.VMEM((NCHUNK, CHUNK), jnp.int32),
            pltpu.VMEM((CHUNK,), jnp.int32),
            pltpu.VMEM((CHUNK, H), jnp.float32),
            pltpu.VMEM_SHARED((N, H), jnp.float32),
        ],
    )
    def k(dst_hbm, zero_hbm, ones_hbm, out_hbm, didx, dbuf, ones_v, acc):
        c = lax.axis_index("c")
        s = lax.axis_index("s")
        wid = s * NC + c
        cbase = wid * NCHUNK
        pltpu.sync_copy(dst_hbm.at[pl.ds(cbase, NCHUNK)], didx)
        pltpu.sync_copy(ones_hbm, ones_v)

        @pl.when(s == 0)
        def _():
            pltpu.sync_copy(zero_hbm, acc)

        plsc.subcore_barrier()

        def body(t, carry):
            for j in range(CHUNK // 16):
                sl = pl.ds(j * 16, 16)
                dbuf[sl] = didx.at[t][sl]
            pltpu.sync_copy(ones_v, acc.at[dbuf], add=True)
            return carry

        lax.fori_loop(0, NCHUNK, body, 0)
        plsc.subcore_barrier()
        rpt = N // NS
        pltpu.sync_copy(acc.at[pl.ds(s * rpt, rpt)],
                        out_hbm.at[c].at[pl.ds(s * rpt, rpt)])

    return k(dst2d, zeros_acc, ones_rows)


def _seg_h(h, src2d, dst2d, zeros_acc):
    mesh = plsc.VectorSubcoreMesh(core_axis_name="c", subcore_axis_name="s")

    @functools.partial(
        pl.kernel,
        out_type=jax.ShapeDtypeStruct((NC, N, H), jnp.float32),
        mesh=mesh,
        scratch_types=[
            pltpu.VMEM((NCHUNK, CHUNK), jnp.int32),
            pltpu.VMEM((NCHUNK, CHUNK), jnp.int32),
            [pltpu.VMEM((CHUNK,), jnp.int32)] * NBUF1,
            [pltpu.VMEM((CHUNK,), jnp.int32)] * NBUF1,
            pltpu.VMEM((NBUF1, CHUNK, H), jnp.float32),
            pltpu.VMEM_SHARED((N, H), jnp.float32),
            [pltpu.SemaphoreType.DMA] * NBUF1,
        ],
    )
    def k(h_hbm, src_hbm, dst_hbm, zero_hbm, out_hbm,
          sidx, didx, sbuf, dbuf, rows, acc, sems):
        c = lax.axis_index("c")
        s = lax.axis_index("s")
        wid = s * NC + c
        cbase = wid * NCHUNK
        pltpu.sync_copy(src_hbm.at[pl.ds(cbase, NCHUNK)], sidx)
        pltpu.sync_copy(dst_hbm.at[pl.ds(cbase, NCHUNK)], didx)

        @pl.when(s == 0)
        def _():
            @pl.when(c == 0)
            def _():
                pltpu.sync_copy(h_hbm, acc)

            @pl.when(c != 0)
            def _():
                pltpu.sync_copy(zero_hbm, acc)

        plsc.subcore_barrier()

        def row_to(buf, src_ref, t):
            for j in range(CHUNK // 16):
                sl = pl.ds(j * 16, 16)
                buf[sl] = src_ref.at[t][sl]

        def fire(t, b):
            row_to(sbuf[b], sidx, t)
            pltpu.async_copy(h_hbm.at[sbuf[b]], rows.at[b], sems[b])

        for b in range(NBUF1):
            fire(b, b)

        def step(t, b):
            pltpu.make_async_copy(h_hbm.at[sbuf[b]], rows.at[b],
                                  sems[b]).wait()
            row_to(dbuf[b], didx, t)
            pltpu.sync_copy(rows.at[b], acc.at[dbuf[b]], add=True)

        def body(jj, carry):
            for b in range(NBUF1):
                t = jj * NBUF1 + b
                step(t, b)
                fire(t + NBUF1, b)
            return carry

        lax.fori_loop(0, (NCHUNK - NBUF1) // NBUF1, body, 0)
        for b in range(NBUF1):
            step(NCHUNK - NBUF1 + b, b)

        plsc.subcore_barrier()
        rpt = N // NS
        pltpu.sync_copy(acc.at[pl.ds(s * rpt, rpt)],
                        out_hbm.at[c].at[pl.ds(s * rpt, rpt)])

    return k(h, src2d, dst2d, zeros_acc)


def _feat_body(h_ref, pdeg_ref, batch_ref, wq_ref, bq_ref, wk_ref,
               bk_ref, wv_ref, bv_ref, wo_ref, q_ref, k_ref, vo_ref):
    h = h_ref[...]
    b = batch_ref[...]
    gsz = jnp.zeros((N, 1), jnp.float32)
    for g in range(NGRAPH):
        m = (b == g).astype(jnp.float32)
        gsz = gsz + m * jnp.sum(m)
    sf1 = jnp.log1p(gsz)
    deg = pdeg_ref[0][:, 0:1] + pdeg_ref[1][:, 0:1]
    sf2 = jnp.log1p(deg)

    def proj(w_ref_, b_ref_):
        w = w_ref_[...]
        return (jnp.dot(h, w[:H, :], preferred_element_type=jnp.float32)
                + sf1 * w[H:H + 1, :] + sf2 * w[H + 1:H + 2, :] + b_ref_[...])

    scale = 1.0 / jnp.sqrt(jnp.float32(ZDIM))
    q_ref[...] = proj(wq_ref, bq_ref) * scale
    k_ref[...] = proj(wk_ref, bk_ref)
    vo_ref[...] = jnp.dot(proj(wv_ref, bv_ref), wo_ref[...],
                          preferred_element_type=jnp.float32)


def _features(h, pdeg, batch2d, Wq, bq, Wk, bk, Wv, bv, Wo):
    f32 = jnp.float32
    return pl.pallas_call(
        _feat_body,
        out_shape=(
            jax.ShapeDtypeStruct((N, H), f32),
            jax.ShapeDtypeStruct((N, H), f32),
            jax.ShapeDtypeStruct((N, NEXP), f32),
        ),
    )(h, pdeg, batch2d, Wq, bq, Wk, bk, Wv, bv, Wo)


def _attn_body(q_ref, k_ref, vo_ref, bo_ref, epk_ref, w1_ref, w2_ref):
    s = lax.dot_general(q_ref[...], k_ref[...], (((1,), (1,)), ((), ())),
                        preferred_element_type=jnp.float32)
    p = jnp.exp(s)
    l = jnp.sum(p, axis=1, keepdims=True)
    logits = (jnp.dot(p, vo_ref[...], preferred_element_type=jnp.float32) / l
              + bo_ref[...])
    lm = jnp.max(logits, axis=1, keepdims=True)
    le = jnp.exp(logits - lm)
    probs = le / jnp.sum(le, axis=1, keepdims=True)

    v1 = jnp.full((BR, 1), -1.0, jnp.float32)
    i1 = jnp.zeros((BR, 1), jnp.int32)
    for e in range(NEXP):
        ce = probs[:, e:e + 1]
        better = ce > v1
        v1 = jnp.where(better, ce, v1)
        i1 = jnp.where(better, e, i1)
    v2 = jnp.full((BR, 1), -1.0, jnp.float32)
    i2 = jnp.zeros((BR, 1), jnp.int32)
    for e in range(NEXP):
        ce = probs[:, e:e + 1]
        better = (ce > v2) & (i1 != e)
        v2 = jnp.where(better, ce, v2)
        i2 = jnp.where(better, e, i2)
    den = v1 + v2 + 1e-9
    epk_ref[...] = i1 + i2 * NEXP
    w1_ref[...] = v1 / den
    w2_ref[...] = v2 / den


def _attention(q, k, vo, bo):
    f32 = jnp.float32
    i32 = jnp.int32
    nb = N // BR
    return pl.pallas_call(
        _attn_body,
        grid=(nb,),
        in_specs=[
            pl.BlockSpec((BR, H), lambda i: (i, 0)),
            pl.BlockSpec((N, H), lambda i: (0, 0)),
            pl.BlockSpec((N, NEXP), lambda i: (0, 0)),
            pl.BlockSpec((1, NEXP), lambda i: (0, 0)),
        ],
        out_specs=(
            pl.BlockSpec((BR, 1), lambda i: (i, 0)),
            pl.BlockSpec((BR, 1), lambda i: (i, 0)),
            pl.BlockSpec((BR, 1), lambda i: (i, 0)),
        ),
        out_shape=(
            jax.ShapeDtypeStruct((N, 1), i32),
            jax.ShapeDtypeStruct((N, 1), f32),
            jax.ShapeDtypeStruct((N, 1), f32),
        ),
    )(q, k, vo, bo)


def _exp_body(parts_ref, we1_ref, be1_ref, we2_ref, epk_ref, w1_ref, w2_ref,
              be2_ref, u_ref, self_ref):
    e = pl.program_id(0)
    t = parts_ref[0] + parts_ref[1]
    he = jnp.maximum(
        jnp.dot(t, we1_ref[0], preferred_element_type=jnp.float32)
        + be1_ref[0], 0.0)
    u = jnp.dot(he, we2_ref[0], preferred_element_type=jnp.float32)
    u_ref[...] = u[None]
    epk = epk_ref[...]
    e1 = jnp.bitwise_and(epk, NEXP - 1)
    e2 = jnp.right_shift(epk, 3)
    gate = (w1_ref[...] * (e1 == e).astype(jnp.float32)
            + w2_ref[...] * (e2 == e).astype(jnp.float32))

    @pl.when(e == 0)
    def _():
        self_ref[...] = jnp.zeros((N, OUT), jnp.float32)

    self_ref[...] += gate * (u + be2_ref[0])


def _experts(parts, We1, be1, We2, epk, w1, w2, be2):
    return pl.pallas_call(
        _exp_body,
        grid=(NEXP,),
        in_specs=[
            pl.BlockSpec((NC, N, H), lambda e: (0, 0, 0)),
            pl.BlockSpec((1, H, H), lambda e: (e, 0, 0)),
            pl.BlockSpec((1, 1, H), lambda e: (e, 0, 0)),
            pl.BlockSpec((1, H, OUT), lambda e: (e, 0, 0)),
            pl.BlockSpec((N, 1), lambda e: (0, 0)),
            pl.BlockSpec((N, 1), lambda e: (0, 0)),
            pl.BlockSpec((N, 1), lambda e: (0, 0)),
            pl.BlockSpec((1, 1, OUT), lambda e: (e, 0, 0)),
        ],
        out_specs=(
            pl.BlockSpec((1, N, OUT), lambda e: (e, 0, 0)),
            pl.BlockSpec((N, OUT), lambda e: (0, 0)),
        ),
        out_shape=(
            jax.ShapeDtypeStruct((NEXP, N, OUT), jnp.float32),
            jax.ShapeDtypeStruct((N, OUT), jnp.float32),
        ),
    )(parts, We1, be1.reshape(NEXP, 1, H), We2, epk, w1, w2,
      be2.reshape(NEXP, 1, OUT))


def _seg_gated(uflat, src2d, dst2d, epk, zeros_nh):
    mesh = plsc.VectorSubcoreMesh(core_axis_name="c", subcore_axis_name="s")

    @functools.partial(
        pl.kernel,
        out_type=(
            jax.ShapeDtypeStruct((NC, N, OUT), jnp.float32),
            jax.ShapeDtypeStruct((NC, N, OUT), jnp.float32),
        ),
        mesh=mesh,
        scratch_types=[
            pltpu.VMEM((N,), jnp.int32),
            pltpu.VMEM((NCHUNK2, CHUNK2), jnp.int32),
            pltpu.VMEM((NCHUNK2, CHUNK2), jnp.int32),
            [pltpu.VMEM((CHUNK2,), jnp.int32)] * NBUF2,
            [pltpu.VMEM((CHUNK2,), jnp.int32)] * NBUF2,
            pltpu.VMEM((NBUF2, CHUNK2, OUT), jnp.float32),
            pltpu.VMEM_SHARED((N, OUT), jnp.float32),
            pltpu.VMEM_SHARED((N, OUT), jnp.float32),
            [pltpu.SemaphoreType.DMA] * NBUF2,
        ],
        compiler_params=pltpu.CompilerParams(needs_layout_passes=False),
    )
    def k(u_hbm, src_hbm, dst_hbm, epk_hbm, zero_hbm, out1_hbm, out2_hbm,
          epkv, sidx, didx, gb, dbuf, rows, acc1, acc2, sems):
        c = lax.axis_index("c")
        s = lax.axis_index("s")
        wid = s * NC + c
        cbase = wid * NCHUNK2
        pltpu.sync_copy(src_hbm.at[pl.ds(cbase, NCHUNK2)], sidx)
        pltpu.sync_copy(dst_hbm.at[pl.ds(cbase, NCHUNK2)], didx)
        pltpu.sync_copy(epk_hbm, epkv)

        @pl.when(s == 0)
        def _():
            pltpu.sync_copy(zero_hbm, acc1)
            pltpu.sync_copy(zero_hbm, acc2)

        plsc.subcore_barrier()

        atab = [acc1, acc2]

        def fire(t_chunk, b):
            for j in range(CHUNK2 // 16):
                sl = pl.ds(j * 16, 16)
                sv = sidx.at[t_chunk][sl]
                dv = didx.at[t_chunk][sl]
                ev = plsc.load_gather(epkv, [dv])
                if b % 2 == 0:
                    es = jnp.bitwise_and(ev, NEXP - 1)
                else:
                    es = jnp.right_shift(ev, 3)
                gb[b][sl] = es * N + sv
            pltpu.async_copy(u_hbm.at[gb[b]], rows.at[b], sems[b])

        def step(t_chunk, b):
            pltpu.make_async_copy(u_hbm.at[gb[b]], rows.at[b],
                                  sems[b]).wait()
            for j in range(CHUNK2 // 16):
                sl = pl.ds(j * 16, 16)
                dbuf[b][sl] = didx.at[t_chunk][sl]
            pltpu.sync_copy(rows.at[b], atab[b % 2].at[dbuf[b]],
                            add=True)

        for b in range(NBUF2):
            fire(b // 2, b)

        def body(jj, carry):
            for b in range(NBUF2):
                t_chunk = jj * (NBUF2 // 2) + b // 2
                step(t_chunk, b)
                fire(t_chunk + NBUF2 // 2, b)
            return carry

        nmain = (2 * NCHUNK2 - NBUF2) // NBUF2
        lax.fori_loop(0, nmain, body, 0)
        for b in range(NBUF2):
            step(NCHUNK2 - NBUF2 // 2 + b // 2, b)

        plsc.subcore_barrier()
        rpt = N // NS
        pltpu.sync_copy(acc1.at[pl.ds(s * rpt, rpt)],
                        out1_hbm.at[c].at[pl.ds(s * rpt, rpt)])
        pltpu.sync_copy(acc2.at[pl.ds(s * rpt, rpt)],
                        out2_hbm.at[c].at[pl.ds(s * rpt, rpt)])

    return k(uflat, src2d, dst2d, epk, zeros_nh)


def _comb_body(self_ref, w1_ref, w2_ref, m1_ref, m2_ref, out_ref):
    m1 = m1_ref[0] + m1_ref[1]
    m2 = m2_ref[0] + m2_ref[1]
    out_ref[...] = self_ref[...] + w1_ref[...] * m1 + w2_ref[...] * m2


def _combine(selft, w1, w2, M1p, M2p):
    nb = N // BC
    return pl.pallas_call(
        _comb_body,
        grid=(nb,),
        in_specs=[
            pl.BlockSpec((BC, OUT), lambda i: (i, 0)),
            pl.BlockSpec((BC, 1), lambda i: (i, 0)),
            pl.BlockSpec((BC, 1), lambda i: (i, 0)),
            pl.BlockSpec((NC, BC, OUT), lambda i: (0, i, 0)),
            pl.BlockSpec((NC, BC, OUT), lambda i: (0, i, 0)),
        ],
        out_specs=pl.BlockSpec((BC, OUT), lambda i: (i, 0)),
        out_shape=jax.ShapeDtypeStruct((N, OUT), jnp.float32),
    )(selft, w1, w2, M1p, M2p)


def kernel(x, edge_index, batch, W_enc, b_enc, Wq, bq, Wk, bk, Wv, bv, Wo, bo,
           We1, be1, We2, be2):
    f32 = jnp.float32
    xs = x[:, 4:10]
    src2d = edge_index[0].reshape(E // CHUNK, CHUNK)
    dst2d = edge_index[1].reshape(E // CHUNK, CHUNK)

    zeros_nh0 = jnp.zeros((N, H), f32)
    ones_rows = jnp.ones((CHUNK, H), f32)
    pdeg = _deg_sc(dst2d, zeros_nh0, ones_rows)
    h = _encode(xs, W_enc, b_enc.reshape(1, H))
    parts = _seg_h(h, src2d, dst2d, zeros_nh0)

    q, k, vo = _features(h, pdeg, batch.reshape(N, 1), Wq,
                         bq.reshape(1, H), Wk, bk.reshape(1, H), Wv,
                         bv.reshape(1, H), Wo)
    epk, w1, w2 = _attention(q, k, vo, bo.reshape(1, NEXP))

    U, selft = _experts(parts, We1, be1, We2, epk, w1, w2, be2)

    src2d64 = edge_index[0].reshape(E // CHUNK2, CHUNK2)
    dst2d64 = edge_index[1].reshape(E // CHUNK2, CHUNK2)
    M1p, M2p = _seg_gated(U.reshape(N * NEXP, OUT), src2d64, dst2d64,
                          epk.reshape(N), zeros_nh0)

    return _combine(selft, w1, w2, M1p, M2p)


# speedup vs baseline: 1.0517x; 1.0517x over previous
import functools

import jax
import jax.numpy as jnp
from jax import lax
from jax.experimental import pallas as pl
from jax.experimental.pallas import tpu as pltpu
from jax.experimental.pallas import tpu_sc as plsc

N = 4096
E = 65536
H = 128
OUT = 128
NEXP = 8
NGRAPH = 8
ZDIM = 130

NC = 2
NS = 16
NW = NC * NS
EPT = E // NW
CHUNK = 128
NCHUNK = EPT // CHUNK
NBUF1 = 4
NBUF2 = 2
CHUNK2 = 128
NCHUNK2 = EPT // CHUNK2

BR = 512
BC = 512
EB = 8192


def _enc_body(xs_ref, w_ref, b_ref, out_ref):
    out_ref[...] = jnp.maximum(
        jnp.dot(xs_ref[...], w_ref[...], preferred_element_type=jnp.float32)
        + b_ref[...], 0.0)


def _encode(xs, W_enc, b_enc):
    return pl.pallas_call(
        _enc_body,
        out_shape=jax.ShapeDtypeStruct((N, H), jnp.float32),
    )(xs, W_enc, b_enc)


def _deg_sc(dst2d, zeros_acc, ones_rows):
    mesh = plsc.VectorSubcoreMesh(core_axis_name="c", subcore_axis_name="s")

    @functools.partial(
        pl.kernel,
        out_type=jax.ShapeDtypeStruct((NC, N, H), jnp.float32),
        mesh=mesh,
        scratch_types=[
            pltpu.VMEM((NCHUNK, CHUNK), jnp.int32),
            pltpu.VMEM((CHUNK,), jnp.int32),
            pltpu.VMEM((CHUNK, H), jnp.float32),
            pltpu.VMEM_SHARED((N, H), jnp.float32),
        ],
    )
    def k(dst_hbm, zero_hbm, ones_hbm, out_hbm, didx, dbuf, ones_v, acc):
        c = lax.axis_index("c")
        s = lax.axis_index("s")
        wid = s * NC + c
        cbase = wid * NCHUNK
        pltpu.sync_copy(dst_hbm.at[pl.ds(cbase, NCHUNK)], didx)
        pltpu.sync_copy(ones_hbm, ones_v)

        @pl.when(s == 0)
        def _():
            pltpu.sync_copy(zero_hbm, acc)

        plsc.subcore_barrier()

        def body(t, carry):
            for j in range(CHUNK // 16):
                sl = pl.ds(j * 16, 16)
                dbuf[sl] = didx.at[t][sl]
            pltpu.sync_copy(ones_v, acc.at[dbuf], add=True)
            return carry

        lax.fori_loop(0, NCHUNK, body, 0)
        plsc.subcore_barrier()
        rpt = N // NS
        pltpu.sync_copy(acc.at[pl.ds(s * rpt, rpt)],
                        out_hbm.at[c].at[pl.ds(s * rpt, rpt)])

    return k(dst2d, zeros_acc, ones_rows)


def _seg_h(h, src2d, dst2d, zeros_acc):
    mesh = plsc.VectorSubcoreMesh(core_axis_name="c", subcore_axis_name="s")

    @functools.partial(
        pl.kernel,
        out_type=jax.ShapeDtypeStruct((NC, N, H), jnp.float32),
        mesh=mesh,
        scratch_types=[
            pltpu.VMEM((NCHUNK, CHUNK), jnp.int32),
            pltpu.VMEM((NCHUNK, CHUNK), jnp.int32),
            [pltpu.VMEM((CHUNK,), jnp.int32)] * NBUF1,
            [pltpu.VMEM((CHUNK,), jnp.int32)] * NBUF1,
            pltpu.VMEM((NBUF1, CHUNK, H), jnp.float32),
            pltpu.VMEM_SHARED((N, H), jnp.float32),
            [pltpu.SemaphoreType.DMA] * NBUF1,
        ],
    )
    def k(h_hbm, src_hbm, dst_hbm, zero_hbm, out_hbm,
          sidx, didx, sbuf, dbuf, rows, acc, sems):
        c = lax.axis_index("c")
        s = lax.axis_index("s")
        wid = s * NC + c
        cbase = wid * NCHUNK
        pltpu.sync_copy(src_hbm.at[pl.ds(cbase, NCHUNK)], sidx)
        pltpu.sync_copy(dst_hbm.at[pl.ds(cbase, NCHUNK)], didx)

        @pl.when(s == 0)
        def _():
            @pl.when(c == 0)
            def _():
                pltpu.sync_copy(h_hbm, acc)

            @pl.when(c != 0)
            def _():
                pltpu.sync_copy(zero_hbm, acc)

        plsc.subcore_barrier()

        def row_to(buf, src_ref, t):
            for j in range(CHUNK // 16):
                sl = pl.ds(j * 16, 16)
                buf[sl] = src_ref.at[t][sl]

        def fire(t, b):
            row_to(sbuf[b], sidx, t)
            pltpu.async_copy(h_hbm.at[sbuf[b]], rows.at[b], sems[b])

        for b in range(NBUF1):
            fire(b, b)

        def step(t, b):
            pltpu.make_async_copy(h_hbm.at[sbuf[b]], rows.at[b],
                                  sems[b]).wait()
            row_to(dbuf[b], didx, t)
            pltpu.sync_copy(rows.at[b], acc.at[dbuf[b]], add=True)

        def body(jj, carry):
            for b in range(NBUF1):
                t = jj * NBUF1 + b
                step(t, b)
                fire(t + NBUF1, b)
            return carry

        lax.fori_loop(0, (NCHUNK - NBUF1) // NBUF1, body, 0)
        for b in range(NBUF1):
            step(NCHUNK - NBUF1 + b, b)

        plsc.subcore_barrier()
        rpt = N // NS
        pltpu.sync_copy(acc.at[pl.ds(s * rpt, rpt)],
                        out_hbm.at[c].at[pl.ds(s * rpt, rpt)])

    return k(h, src2d, dst2d, zeros_acc)


def _feat_body(h_ref, pdeg_ref, batch_ref, wq_ref, bq_ref, wk_ref,
               bk_ref, wv_ref, bv_ref, wo_ref, q_ref, k_ref, vo_ref):
    h = h_ref[...]
    b = batch_ref[...]
    gsz = jnp.zeros((N, 1), jnp.float32)
    for g in range(NGRAPH):
        m = (b == g).astype(jnp.float32)
        gsz = gsz + m * jnp.sum(m)
    sf1 = jnp.log1p(gsz)
    deg = pdeg_ref[0][:, 0:1] + pdeg_ref[1][:, 0:1]
    sf2 = jnp.log1p(deg)

    def proj(w_ref_, b_ref_):
        w = w_ref_[...]
        return (jnp.dot(h, w[:H, :], preferred_element_type=jnp.float32)
                + sf1 * w[H:H + 1, :] + sf2 * w[H + 1:H + 2, :] + b_ref_[...])

    scale = 1.0 / jnp.sqrt(jnp.float32(ZDIM))
    q_ref[...] = proj(wq_ref, bq_ref) * scale
    k_ref[...] = proj(wk_ref, bk_ref)
    vo_ref[...] = jnp.dot(proj(wv_ref, bv_ref), wo_ref[...],
                          preferred_element_type=jnp.float32)


def _features(h, pdeg, batch2d, Wq, bq, Wk, bk, Wv, bv, Wo):
    f32 = jnp.float32
    return pl.pallas_call(
        _feat_body,
        out_shape=(
            jax.ShapeDtypeStruct((N, H), f32),
            jax.ShapeDtypeStruct((N, H), f32),
            jax.ShapeDtypeStruct((N, NEXP), f32),
        ),
    )(h, pdeg, batch2d, Wq, bq, Wk, bk, Wv, bv, Wo)


def _attn_body(q_ref, k_ref, vo_ref, bo_ref, epk_ref, w1_ref, w2_ref):
    s = lax.dot_general(q_ref[...], k_ref[...], (((1,), (1,)), ((), ())),
                        preferred_element_type=jnp.float32)
    p = jnp.exp(s)
    l = jnp.sum(p, axis=1, keepdims=True)
    logits = (jnp.dot(p, vo_ref[...], preferred_element_type=jnp.float32) / l
              + bo_ref[...])
    lm = jnp.max(logits, axis=1, keepdims=True)
    le = jnp.exp(logits - lm)
    probs = le / jnp.sum(le, axis=1, keepdims=True)

    v1 = jnp.full((BR, 1), -1.0, jnp.float32)
    i1 = jnp.zeros((BR, 1), jnp.int32)
    for e in range(NEXP):
        ce = probs[:, e:e + 1]
        better = ce > v1
        v1 = jnp.where(better, ce, v1)
        i1 = jnp.where(better, e, i1)
    v2 = jnp.full((BR, 1), -1.0, jnp.float32)
    i2 = jnp.zeros((BR, 1), jnp.int32)
    for e in range(NEXP):
        ce = probs[:, e:e + 1]
        better = (ce > v2) & (i1 != e)
        v2 = jnp.where(better, ce, v2)
        i2 = jnp.where(better, e, i2)
    den = v1 + v2 + 1e-9
    epk_ref[...] = i1 + i2 * NEXP
    w1_ref[...] = v1 / den
    w2_ref[...] = v2 / den


def _attention(q, k, vo, bo):
    f32 = jnp.float32
    i32 = jnp.int32
    nb = N // BR
    return pl.pallas_call(
        _attn_body,
        grid=(nb,),
        in_specs=[
            pl.BlockSpec((BR, H), lambda i: (i, 0)),
            pl.BlockSpec((N, H), lambda i: (0, 0)),
            pl.BlockSpec((N, NEXP), lambda i: (0, 0)),
            pl.BlockSpec((1, NEXP), lambda i: (0, 0)),
        ],
        out_specs=(
            pl.BlockSpec((BR, 1), lambda i: (i, 0)),
            pl.BlockSpec((BR, 1), lambda i: (i, 0)),
            pl.BlockSpec((BR, 1), lambda i: (i, 0)),
        ),
        out_shape=(
            jax.ShapeDtypeStruct((N, 1), i32),
            jax.ShapeDtypeStruct((N, 1), f32),
            jax.ShapeDtypeStruct((N, 1), f32),
        ),
    )(q, k, vo, bo)


def _exp_body(parts_ref, we1_ref, be1_ref, we2_ref, u_ref):
    t = parts_ref[0] + parts_ref[1]
    he = jnp.maximum(
        jnp.dot(t, we1_ref[0], preferred_element_type=jnp.float32)
        + be1_ref[0], 0.0)
    u = jnp.dot(he, we2_ref[0], preferred_element_type=jnp.float32)
    u_ref[...] = u[None]


def _experts(parts, We1, be1, We2):
    return pl.pallas_call(
        _exp_body,
        grid=(NEXP,),
        in_specs=[
            pl.BlockSpec((NC, N, H), lambda e: (0, 0, 0)),
            pl.BlockSpec((1, H, H), lambda e: (e, 0, 0)),
            pl.BlockSpec((1, 1, H), lambda e: (e, 0, 0)),
            pl.BlockSpec((1, H, OUT), lambda e: (e, 0, 0)),
        ],
        out_specs=pl.BlockSpec((1, N, OUT), lambda e: (e, 0, 0)),
        out_shape=jax.ShapeDtypeStruct((NEXP, N, OUT), jnp.float32),
    )(parts, We1, be1.reshape(NEXP, 1, H), We2)


def _seg_gated(uflat, src2d, dst2d, epk, zeros_nh):
    mesh = plsc.VectorSubcoreMesh(core_axis_name="c", subcore_axis_name="s")

    @functools.partial(
        pl.kernel,
        out_type=(
            jax.ShapeDtypeStruct((NC, N, OUT), jnp.float32),
            jax.ShapeDtypeStruct((NC, N, OUT), jnp.float32),
        ),
        mesh=mesh,
        scratch_types=[
            pltpu.VMEM((N,), jnp.int32),
            pltpu.VMEM((NCHUNK2, CHUNK2), jnp.int32),
            pltpu.VMEM((NCHUNK2, CHUNK2), jnp.int32),
            [pltpu.VMEM((CHUNK2,), jnp.int32)] * NBUF2,
            [pltpu.VMEM((CHUNK2,), jnp.int32)] * NBUF2,
            pltpu.VMEM((NBUF2, CHUNK2, OUT), jnp.float32),
            pltpu.VMEM_SHARED((N, OUT), jnp.float32),
            pltpu.VMEM_SHARED((N, OUT), jnp.float32),
            [pltpu.SemaphoreType.DMA] * NBUF2,
        ],
        compiler_params=pltpu.CompilerParams(needs_layout_passes=False),
    )
    def k(u_hbm, src_hbm, dst_hbm, epk_hbm, zero_hbm, out1_hbm, out2_hbm,
          epkv, sidx, didx, gb, dbuf, rows, acc1, acc2, sems):
        c = lax.axis_index("c")
        s = lax.axis_index("s")
        wid = s * NC + c
        cbase = wid * NCHUNK2
        pltpu.sync_copy(src_hbm.at[pl.ds(cbase, NCHUNK2)], sidx)
        pltpu.sync_copy(dst_hbm.at[pl.ds(cbase, NCHUNK2)], didx)
        pltpu.sync_copy(epk_hbm, epkv)

        @pl.when(s == 0)
        def _():
            pltpu.sync_copy(zero_hbm, acc1)
            pltpu.sync_copy(zero_hbm, acc2)

        plsc.subcore_barrier()

        atab = [acc1, acc2]

        def fire(t_chunk, b):
            for j in range(CHUNK2 // 16):
                sl = pl.ds(j * 16, 16)
                sv = sidx.at[t_chunk][sl]
                dv = didx.at[t_chunk][sl]
                ev = plsc.load_gather(epkv, [dv])
                if b % 2 == 0:
                    es = jnp.bitwise_and(ev, NEXP - 1)
                else:
                    es = jnp.right_shift(ev, 3)
                gb[b][sl] = es * N + sv
            pltpu.async_copy(u_hbm.at[gb[b]], rows.at[b], sems[b])

        def step(t_chunk, b):
            pltpu.make_async_copy(u_hbm.at[gb[b]], rows.at[b],
                                  sems[b]).wait()
            for j in range(CHUNK2 // 16):
                sl = pl.ds(j * 16, 16)
                dbuf[b][sl] = didx.at[t_chunk][sl]
            pltpu.sync_copy(rows.at[b], atab[b % 2].at[dbuf[b]],
                            add=True)

        for b in range(NBUF2):
            fire(b // 2, b)

        def body(jj, carry):
            for b in range(NBUF2):
                t_chunk = jj * (NBUF2 // 2) + b // 2
                step(t_chunk, b)
                fire(t_chunk + NBUF2 // 2, b)
            return carry

        nmain = (2 * NCHUNK2 - NBUF2) // NBUF2
        lax.fori_loop(0, nmain, body, 0)
        for b in range(NBUF2):
            step(NCHUNK2 - NBUF2 // 2 + b // 2, b)

        plsc.subcore_barrier()
        rpt = N // NS
        pltpu.sync_copy(acc1.at[pl.ds(s * rpt, rpt)],
                        out1_hbm.at[c].at[pl.ds(s * rpt, rpt)])
        pltpu.sync_copy(acc2.at[pl.ds(s * rpt, rpt)],
                        out2_hbm.at[c].at[pl.ds(s * rpt, rpt)])

    return k(uflat, src2d, dst2d, epk, zeros_nh)


def _comb_body(u_ref, epk_ref, w1_ref, w2_ref, m1_ref, m2_ref,
               be2_ref, out_ref):
    u = u_ref[...]
    epk = epk_ref[...]
    e1 = jnp.bitwise_and(epk, NEXP - 1)
    e2 = jnp.right_shift(epk, 3)
    sel1 = jnp.zeros((BC, OUT), jnp.float32)
    sel2 = jnp.zeros((BC, OUT), jnp.float32)
    be2 = be2_ref[...]
    for e in range(NEXP):
        ue = u[e] + be2[e:e + 1, :]
        sel1 = sel1 + (e1 == e).astype(jnp.float32) * ue
        sel2 = sel2 + (e2 == e).astype(jnp.float32) * ue
    m1 = m1_ref[0] + m1_ref[1]
    m2 = m2_ref[0] + m2_ref[1]
    out_ref[...] = w1_ref[...] * (sel1 + m1) + w2_ref[...] * (sel2 + m2)


def _combine(U, epk, w1, w2, M1p, M2p, be2):
    nb = N // BC
    return pl.pallas_call(
        _comb_body,
        grid=(nb,),
        in_specs=[
            pl.BlockSpec((NEXP, BC, OUT), lambda i: (0, i, 0)),
            pl.BlockSpec((BC, 1), lambda i: (i, 0)),
            pl.BlockSpec((BC, 1), lambda i: (i, 0)),
            pl.BlockSpec((BC, 1), lambda i: (i, 0)),
            pl.BlockSpec((NC, BC, OUT), lambda i: (0, i, 0)),
            pl.BlockSpec((NC, BC, OUT), lambda i: (0, i, 0)),
            pl.BlockSpec((NEXP, OUT), lambda i: (0, 0)),
        ],
        out_specs=pl.BlockSpec((BC, OUT), lambda i: (i, 0)),
        out_shape=jax.ShapeDtypeStruct((N, OUT), jnp.float32),
    )(U, epk, w1, w2, M1p, M2p, be2)


def kernel(x, edge_index, batch, W_enc, b_enc, Wq, bq, Wk, bk, Wv, bv, Wo, bo,
           We1, be1, We2, be2):
    f32 = jnp.float32
    xs = x[:, 4:10]
    src2d = edge_index[0].reshape(E // CHUNK, CHUNK)
    dst2d = edge_index[1].reshape(E // CHUNK, CHUNK)

    zeros_nh0 = jnp.zeros((N, H), f32)
    ones_rows = jnp.ones((CHUNK, H), f32)
    pdeg = _deg_sc(dst2d, zeros_nh0, ones_rows)
    h = _encode(xs, W_enc, b_enc.reshape(1, H))
    parts = _seg_h(h, src2d, dst2d, zeros_nh0)

    q, k, vo = _features(h, pdeg, batch.reshape(N, 1), Wq,
                         bq.reshape(1, H), Wk, bk.reshape(1, H), Wv,
                         bv.reshape(1, H), Wo)
    epk, w1, w2 = _attention(q, k, vo, bo.reshape(1, NEXP))

    U = _experts(parts, We1, be1, We2)

    M1p, M2p = _seg_gated(U.reshape(N * NEXP, OUT), src2d, dst2d,
                          epk.reshape(N), zeros_nh0)

    return _combine(U, epk, w1, w2, M1p, M2p, be2)


# speedup vs baseline: 1.0853x; 1.0319x over previous
import functools

import jax
import jax.numpy as jnp
from jax import lax
from jax.experimental import pallas as pl
from jax.experimental.pallas import tpu as pltpu
from jax.experimental.pallas import tpu_sc as plsc

N = 4096
E = 65536
H = 128
OUT = 128
NEXP = 8
NGRAPH = 8
ZDIM = 130

NC = 2
NS = 16
NW = NC * NS
EPT = E // NW
CHUNK = 128
NCHUNK = EPT // CHUNK
NBUF1 = 4
NBUF2 = 2
CHUNK2 = 128
NCHUNK2 = EPT // CHUNK2

BR = 512
BC = 1024
EB = 8192


def _enc_body(xs_ref, w_ref, b_ref, out_ref):
    out_ref[...] = jnp.maximum(
        jnp.dot(xs_ref[...], w_ref[...], preferred_element_type=jnp.float32)
        + b_ref[...], 0.0)


def _encode(xs, W_enc, b_enc):
    return pl.pallas_call(
        _enc_body,
        out_shape=jax.ShapeDtypeStruct((N, H), jnp.float32),
    )(xs, W_enc, b_enc)


def _deg_sc(dst2d, zeros_acc, ones_rows):
    mesh = plsc.VectorSubcoreMesh(core_axis_name="c", subcore_axis_name="s")

    @functools.partial(
        pl.kernel,
        out_type=jax.ShapeDtypeStruct((NC, N, H), jnp.float32),
        mesh=mesh,
        scratch_types=[
            pltpu.VMEM((NCHUNK, CHUNK), jnp.int32),
            pltpu.VMEM((CHUNK,), jnp.int32),
            pltpu.VMEM((CHUNK, H), jnp.float32),
            pltpu.VMEM_SHARED((N, H), jnp.float32),
        ],
    )
    def k(dst_hbm, zero_hbm, ones_hbm, out_hbm, didx, dbuf, ones_v, acc):
        c = lax.axis_index("c")
        s = lax.axis_index("s")
        wid = s * NC + c
        cbase = wid * NCHUNK
        pltpu.sync_copy(dst_hbm.at[pl.ds(cbase, NCHUNK)], didx)
        pltpu.sync_copy(ones_hbm, ones_v)

        @pl.when(s == 0)
        def _():
            pltpu.sync_copy(zero_hbm, acc)

        plsc.subcore_barrier()

        def body(t, carry):
            for j in range(CHUNK // 16):
                sl = pl.ds(j * 16, 16)
                dbuf[sl] = didx.at[t][sl]
            pltpu.sync_copy(ones_v, acc.at[dbuf], add=True)
            return carry

        lax.fori_loop(0, NCHUNK, body, 0)
        plsc.subcore_barrier()
        rpt = N // NS
        pltpu.sync_copy(acc.at[pl.ds(s * rpt, rpt)],
                        out_hbm.at[c].at[pl.ds(s * rpt, rpt)])

    return k(dst2d, zeros_acc, ones_rows)


def _seg_h(h, src2d, dst2d, zeros_acc):
    mesh = plsc.VectorSubcoreMesh(core_axis_name="c", subcore_axis_name="s")

    @functools.partial(
        pl.kernel,
        out_type=jax.ShapeDtypeStruct((NC, N, H), jnp.float32),
        mesh=mesh,
        scratch_types=[
            pltpu.VMEM((NCHUNK, CHUNK), jnp.int32),
            pltpu.VMEM((NCHUNK, CHUNK), jnp.int32),
            [pltpu.VMEM((CHUNK,), jnp.int32)] * NBUF1,
            [pltpu.VMEM((CHUNK,), jnp.int32)] * NBUF1,
            pltpu.VMEM((NBUF1, CHUNK, H), jnp.float32),
            pltpu.VMEM_SHARED((N, H), jnp.float32),
            [pltpu.SemaphoreType.DMA] * NBUF1,
        ],
    )
    def k(h_hbm, src_hbm, dst_hbm, zero_hbm, out_hbm,
          sidx, didx, sbuf, dbuf, rows, acc, sems):
        c = lax.axis_index("c")
        s = lax.axis_index("s")
        wid = s * NC + c
        cbase = wid * NCHUNK
        pltpu.sync_copy(src_hbm.at[pl.ds(cbase, NCHUNK)], sidx)
        pltpu.sync_copy(dst_hbm.at[pl.ds(cbase, NCHUNK)], didx)

        @pl.when(s == 0)
        def _():
            @pl.when(c == 0)
            def _():
                pltpu.sync_copy(h_hbm, acc)

            @pl.when(c != 0)
            def _():
                pltpu.sync_copy(zero_hbm, acc)

        plsc.subcore_barrier()

        def row_to(buf, src_ref, t):
            for j in range(CHUNK // 16):
                sl = pl.ds(j * 16, 16)
                buf[sl] = src_ref.at[t][sl]

        def fire(t, b):
            row_to(sbuf[b], sidx, t)
            pltpu.async_copy(h_hbm.at[sbuf[b]], rows.at[b], sems[b])

        for b in range(NBUF1):
            fire(b, b)

        def step(t, b):
            pltpu.make_async_copy(h_hbm.at[sbuf[b]], rows.at[b],
                                  sems[b]).wait()
            row_to(dbuf[b], didx, t)
            pltpu.sync_copy(rows.at[b], acc.at[dbuf[b]], add=True)

        def body(jj, carry):
            for b in range(NBUF1):
                t = jj * NBUF1 + b
                step(t, b)
                fire(t + NBUF1, b)
            return carry

        lax.fori_loop(0, (NCHUNK - NBUF1) // NBUF1, body, 0)
        for b in range(NBUF1):
            step(NCHUNK - NBUF1 + b, b)

        plsc.subcore_barrier()
        rpt = N // NS
        pltpu.sync_copy(acc.at[pl.ds(s * rpt, rpt)],
                        out_hbm.at[c].at[pl.ds(s * rpt, rpt)])

    return k(h, src2d, dst2d, zeros_acc)


def _featm_body(h_ref, batch_ref, wq_ref, bq_ref, wk_ref, bk_ref, wv_ref,
                bv_ref, wo_ref, qm_ref, km_ref, vom_ref):
    h = h_ref[...]
    b = batch_ref[...]
    gsz = jnp.zeros((N, 1), jnp.float32)
    for g in range(NGRAPH):
        m = (b == g).astype(jnp.float32)
        gsz = gsz + m * jnp.sum(m)
    sf1 = jnp.log1p(gsz)

    def proj(w_ref_, b_ref_):
        w = w_ref_[...]
        return (jnp.dot(h, w[:H, :], preferred_element_type=jnp.float32)
                + sf1 * w[H:H + 1, :] + b_ref_[...])

    qm_ref[...] = proj(wq_ref, bq_ref)
    km_ref[...] = proj(wk_ref, bk_ref)
    vom_ref[...] = jnp.dot(proj(wv_ref, bv_ref), wo_ref[...],
                           preferred_element_type=jnp.float32)


def _featm(h, batch2d, Wq, bq, Wk, bk, Wv, bv, Wo):
    f32 = jnp.float32
    return pl.pallas_call(
        _featm_body,
        out_shape=(
            jax.ShapeDtypeStruct((N, H), f32),
            jax.ShapeDtypeStruct((N, H), f32),
            jax.ShapeDtypeStruct((N, NEXP), f32),
        ),
    )(h, batch2d, Wq, bq, Wk, bk, Wv, bv, Wo)


def _featd_body(qm_ref, km_ref, vom_ref, pdeg_ref, wq_ref, wk_ref, wv_ref,
                wo_ref, q_ref, k_ref, vo_ref):
    deg = pdeg_ref[0][:, 0:1] + pdeg_ref[1][:, 0:1]
    sf2 = jnp.log1p(deg)
    scale = 1.0 / jnp.sqrt(jnp.float32(ZDIM))
    q_ref[...] = (qm_ref[...] + sf2 * wq_ref[H + 1:H + 2, :]) * scale
    k_ref[...] = km_ref[...] + sf2 * wk_ref[H + 1:H + 2, :]
    wvo = jnp.dot(wv_ref[H + 1:H + 2, :], wo_ref[...],
                  preferred_element_type=jnp.float32)
    vo_ref[...] = vom_ref[...] + sf2 * wvo


def _featd(qm, km, vom, pdeg, Wq, Wk, Wv, Wo):
    f32 = jnp.float32
    return pl.pallas_call(
        _featd_body,
        out_shape=(
            jax.ShapeDtypeStruct((N, H), f32),
            jax.ShapeDtypeStruct((N, H), f32),
            jax.ShapeDtypeStruct((N, NEXP), f32),
        ),
    )(qm, km, vom, pdeg, Wq, Wk, Wv, Wo)


def _attn_body(q_ref, k_ref, vo_ref, bo_ref, epk_ref, w1_ref, w2_ref):
    s = lax.dot_general(q_ref[...], k_ref[...], (((1,), (1,)), ((), ())),
                        preferred_element_type=jnp.float32)
    p = jnp.exp(s)
    l = jnp.sum(p, axis=1, keepdims=True)
    logits = (jnp.dot(p, vo_ref[...], preferred_element_type=jnp.float32) / l
              + bo_ref[...])
    lm = jnp.max(logits, axis=1, keepdims=True)
    le = jnp.exp(logits - lm)
    probs = le / jnp.sum(le, axis=1, keepdims=True)

    v1 = jnp.full((BR, 1), -1.0, jnp.float32)
    i1 = jnp.zeros((BR, 1), jnp.int32)
    for e in range(NEXP):
        ce = probs[:, e:e + 1]
        better = ce > v1
        v1 = jnp.where(better, ce, v1)
        i1 = jnp.where(better, e, i1)
    v2 = jnp.full((BR, 1), -1.0, jnp.float32)
    i2 = jnp.zeros((BR, 1), jnp.int32)
    for e in range(NEXP):
        ce = probs[:, e:e + 1]
        better = (ce > v2) & (i1 != e)
        v2 = jnp.where(better, ce, v2)
        i2 = jnp.where(better, e, i2)
    den = v1 + v2 + 1e-9
    epk_ref[...] = i1 + i2 * NEXP
    w1_ref[...] = v1 / den
    w2_ref[...] = v2 / den


def _attention(q, k, vo, bo):
    f32 = jnp.float32
    i32 = jnp.int32
    nb = N // BR
    return pl.pallas_call(
        _attn_body,
        grid=(nb,),
        in_specs=[
            pl.BlockSpec((BR, H), lambda i: (i, 0)),
            pl.BlockSpec((N, H), lambda i: (0, 0)),
            pl.BlockSpec((N, NEXP), lambda i: (0, 0)),
            pl.BlockSpec((1, NEXP), lambda i: (0, 0)),
        ],
        out_specs=(
            pl.BlockSpec((BR, 1), lambda i: (i, 0)),
            pl.BlockSpec((BR, 1), lambda i: (i, 0)),
            pl.BlockSpec((BR, 1), lambda i: (i, 0)),
        ),
        out_shape=(
            jax.ShapeDtypeStruct((N, 1), i32),
            jax.ShapeDtypeStruct((N, 1), f32),
            jax.ShapeDtypeStruct((N, 1), f32),
        ),
    )(q, k, vo, bo)


def _exp_body(parts_ref, we1_ref, be1_ref, we2_ref, u_ref):
    t = parts_ref[0] + parts_ref[1]
    he = jnp.maximum(
        jnp.dot(t, we1_ref[0], preferred_element_type=jnp.float32)
        + be1_ref[0], 0.0)
    u = jnp.dot(he, we2_ref[0], preferred_element_type=jnp.float32)
    u_ref[...] = u[None]


def _experts(parts, We1, be1, We2):
    return pl.pallas_call(
        _exp_body,
        grid=(NEXP,),
        in_specs=[
            pl.BlockSpec((NC, N, H), lambda e: (0, 0, 0)),
            pl.BlockSpec((1, H, H), lambda e: (e, 0, 0)),
            pl.BlockSpec((1, 1, H), lambda e: (e, 0, 0)),
            pl.BlockSpec((1, H, OUT), lambda e: (e, 0, 0)),
        ],
        out_specs=pl.BlockSpec((1, N, OUT), lambda e: (e, 0, 0)),
        out_shape=jax.ShapeDtypeStruct((NEXP, N, OUT), jnp.float32),
    )(parts, We1, be1.reshape(NEXP, 1, H), We2)


def _seg_gated(uflat, src2d, dst2d, epk, zeros_nh):
    mesh = plsc.VectorSubcoreMesh(core_axis_name="c", subcore_axis_name="s")

    @functools.partial(
        pl.kernel,
        out_type=(
            jax.ShapeDtypeStruct((NC, N, OUT), jnp.float32),
            jax.ShapeDtypeStruct((NC, N, OUT), jnp.float32),
        ),
        mesh=mesh,
        scratch_types=[
            pltpu.VMEM((N,), jnp.int32),
            pltpu.VMEM((NCHUNK2, CHUNK2), jnp.int32),
            pltpu.VMEM((NCHUNK2, CHUNK2), jnp.int32),
            [pltpu.VMEM((CHUNK2,), jnp.int32)] * NBUF2,
            [pltpu.VMEM((CHUNK2,), jnp.int32)] * NBUF2,
            pltpu.VMEM((NBUF2, CHUNK2, OUT), jnp.float32),
            pltpu.VMEM_SHARED((N, OUT), jnp.float32),
            pltpu.VMEM_SHARED((N, OUT), jnp.float32),
            [pltpu.SemaphoreType.DMA] * NBUF2,
        ],
        compiler_params=pltpu.CompilerParams(needs_layout_passes=False),
    )
    def k(u_hbm, src_hbm, dst_hbm, epk_hbm, zero_hbm, out1_hbm, out2_hbm,
          epkv, sidx, didx, gb, dbuf, rows, acc1, acc2, sems):
        c = lax.axis_index("c")
        s = lax.axis_index("s")
        wid = s * NC + c
        cbase = wid * NCHUNK2
        pltpu.sync_copy(src_hbm.at[pl.ds(cbase, NCHUNK2)], sidx)
        pltpu.sync_copy(dst_hbm.at[pl.ds(cbase, NCHUNK2)], didx)
        pltpu.sync_copy(epk_hbm, epkv)

        @pl.when(s == 0)
        def _():
            pltpu.sync_copy(zero_hbm, acc1)
            pltpu.sync_copy(zero_hbm, acc2)

        plsc.subcore_barrier()

        atab = [acc1, acc2]

        def fire(t_chunk, b):
            for j in range(CHUNK2 // 16):
                sl = pl.ds(j * 16, 16)
                sv = sidx.at[t_chunk][sl]
                dv = didx.at[t_chunk][sl]
                ev = plsc.load_gather(epkv, [dv])
                if b % 2 == 0:
                    es = jnp.bitwise_and(ev, NEXP - 1)
                else:
                    es = jnp.right_shift(ev, 3)
                gb[b][sl] = es * N + sv
            pltpu.async_copy(u_hbm.at[gb[b]], rows.at[b], sems[b])

        def step(t_chunk, b):
            pltpu.make_async_copy(u_hbm.at[gb[b]], rows.at[b],
                                  sems[b]).wait()
            for j in range(CHUNK2 // 16):
                sl = pl.ds(j * 16, 16)
                dbuf[b][sl] = didx.at[t_chunk][sl]
            pltpu.sync_copy(rows.at[b], atab[b % 2].at[dbuf[b]],
                            add=True)

        for b in range(NBUF2):
            fire(b // 2, b)

        def body(jj, carry):
            for b in range(NBUF2):
                t_chunk = jj * (NBUF2 // 2) + b // 2
                step(t_chunk, b)
                fire(t_chunk + NBUF2 // 2, b)
            return carry

        nmain = (2 * NCHUNK2 - NBUF2) // NBUF2
        lax.fori_loop(0, nmain, body, 0)
        for b in range(NBUF2):
            step(NCHUNK2 - NBUF2 // 2 + b // 2, b)

        plsc.subcore_barrier()
        rpt = N // NS
        pltpu.sync_copy(acc1.at[pl.ds(s * rpt, rpt)],
                        out1_hbm.at[c].at[pl.ds(s * rpt, rpt)])
        pltpu.sync_copy(acc2.at[pl.ds(s * rpt, rpt)],
                        out2_hbm.at[c].at[pl.ds(s * rpt, rpt)])

    return k(uflat, src2d, dst2d, epk, zeros_nh)


def _comb_body(u_ref, epk_ref, w1_ref, w2_ref, m1_ref, m2_ref,
               be2_ref, out_ref):
    u = u_ref[...]
    epk = epk_ref[...]
    e1 = jnp.bitwise_and(epk, NEXP - 1)
    e2 = jnp.right_shift(epk, 3)
    sel1 = jnp.zeros((BC, OUT), jnp.float32)
    sel2 = jnp.zeros((BC, OUT), jnp.float32)
    be2 = be2_ref[...]
    for e in range(NEXP):
        ue = u[e] + be2[e:e + 1, :]
        sel1 = sel1 + (e1 == e).astype(jnp.float32) * ue
        sel2 = sel2 + (e2 == e).astype(jnp.float32) * ue
    m1 = m1_ref[0] + m1_ref[1]
    m2 = m2_ref[0] + m2_ref[1]
    out_ref[...] = w1_ref[...] * (sel1 + m1) + w2_ref[...] * (sel2 + m2)


def _combine(U, epk, w1, w2, M1p, M2p, be2):
    nb = N // BC
    return pl.pallas_call(
        _comb_body,
        grid=(nb,),
        in_specs=[
            pl.BlockSpec((NEXP, BC, OUT), lambda i: (0, i, 0)),
            pl.BlockSpec((BC, 1), lambda i: (i, 0)),
            pl.BlockSpec((BC, 1), lambda i: (i, 0)),
            pl.BlockSpec((BC, 1), lambda i: (i, 0)),
            pl.BlockSpec((NC, BC, OUT), lambda i: (0, i, 0)),
            pl.BlockSpec((NC, BC, OUT), lambda i: (0, i, 0)),
            pl.BlockSpec((NEXP, OUT), lambda i: (0, 0)),
        ],
        out_specs=pl.BlockSpec((BC, OUT), lambda i: (i, 0)),
        out_shape=jax.ShapeDtypeStruct((N, OUT), jnp.float32),
    )(U, epk, w1, w2, M1p, M2p, be2)


def kernel(x, edge_index, batch, W_enc, b_enc, Wq, bq, Wk, bk, Wv, bv, Wo, bo,
           We1, be1, We2, be2):
    f32 = jnp.float32
    xs = x[:, 4:10]
    src2d = edge_index[0].reshape(E // CHUNK, CHUNK)
    dst2d = edge_index[1].reshape(E // CHUNK, CHUNK)

    zeros_nh0 = jnp.zeros((N, H), f32)
    ones_rows = jnp.ones((CHUNK, H), f32)
    pdeg = _deg_sc(dst2d, zeros_nh0, ones_rows)
    h = _encode(xs, W_enc, b_enc.reshape(1, H))
    parts = _seg_h(h, src2d, dst2d, zeros_nh0)

    qm, km, vom = _featm(h, batch.reshape(N, 1), Wq, bq.reshape(1, H),
                         Wk, bk.reshape(1, H), Wv, bv.reshape(1, H), Wo)
    q, k, vo = _featd(qm, km, vom, pdeg, Wq, Wk, Wv, Wo)
    epk, w1, w2 = _attention(q, k, vo, bo.reshape(1, NEXP))

    U = _experts(parts, We1, be1, We2)

    M1p, M2p = _seg_gated(U.reshape(N * NEXP, OUT), src2d, dst2d,
                          epk.reshape(N), zeros_nh0)

    return _combine(U, epk, w1, w2, M1p, M2p, be2)


# speedup vs baseline: 1.1468x; 1.0566x over previous
import functools

import jax
import jax.numpy as jnp
from jax import lax
from jax.experimental import pallas as pl
from jax.experimental.pallas import tpu as pltpu
from jax.experimental.pallas import tpu_sc as plsc

N = 4096
E = 65536
H = 128
OUT = 128
NEXP = 8
NGRAPH = 8
ZDIM = 130

NC = 2
NS = 16
NW = NC * NS
EPT = E // NW
CHUNK = 128
NCHUNK = EPT // CHUNK
NBUF1 = 4
NBUF2 = 2
CHUNK2 = 128
NCHUNK2 = EPT // CHUNK2

BR = 512
BC = 1024
EB = 8192


def _enc_body(x_ref, w_ref, b_ref, out_ref):
    xs = x_ref[:, 4:10]
    out_ref[...] = jnp.maximum(
        jnp.dot(xs, w_ref[...], preferred_element_type=jnp.float32)
        + b_ref[...], 0.0)


def _encode(xs, W_enc, b_enc):
    return pl.pallas_call(
        _enc_body,
        out_shape=jax.ShapeDtypeStruct((N, H), jnp.float32),
    )(xs, W_enc, b_enc)


def _deg_sc(dst2d, zeros_acc, ones_rows):
    mesh = plsc.VectorSubcoreMesh(core_axis_name="c", subcore_axis_name="s")

    @functools.partial(
        pl.kernel,
        out_type=jax.ShapeDtypeStruct((NC, N, H), jnp.float32),
        mesh=mesh,
        scratch_types=[
            pltpu.VMEM((NCHUNK, CHUNK), jnp.int32),
            pltpu.VMEM((CHUNK,), jnp.int32),
            pltpu.VMEM((CHUNK, H), jnp.float32),
            pltpu.VMEM_SHARED((N, H), jnp.float32),
        ],
    )
    def k(dst_hbm, zero_hbm, ones_hbm, out_hbm, didx, dbuf, ones_v, acc):
        c = lax.axis_index("c")
        s = lax.axis_index("s")
        wid = s * NC + c
        cbase = wid * NCHUNK
        pltpu.sync_copy(dst_hbm.at[pl.ds(cbase, NCHUNK)], didx)
        pltpu.sync_copy(ones_hbm, ones_v)

        @pl.when(s == 0)
        def _():
            pltpu.sync_copy(zero_hbm, acc)

        plsc.subcore_barrier()

        def body(t, carry):
            for j in range(CHUNK // 16):
                sl = pl.ds(j * 16, 16)
                dbuf[sl] = didx.at[t][sl]
            pltpu.sync_copy(ones_v, acc.at[dbuf], add=True)
            return carry

        lax.fori_loop(0, NCHUNK, body, 0)
        plsc.subcore_barrier()
        rpt = N // NS
        pltpu.sync_copy(acc.at[pl.ds(s * rpt, rpt)],
                        out_hbm.at[c].at[pl.ds(s * rpt, rpt)])

    return k(dst2d, zeros_acc, ones_rows)


def _seg_h(h, src2d, dst2d, zeros_acc):
    mesh = plsc.VectorSubcoreMesh(core_axis_name="c", subcore_axis_name="s")

    @functools.partial(
        pl.kernel,
        out_type=jax.ShapeDtypeStruct((NC, N, H), jnp.float32),
        mesh=mesh,
        scratch_types=[
            pltpu.VMEM((NCHUNK, CHUNK), jnp.int32),
            pltpu.VMEM((NCHUNK, CHUNK), jnp.int32),
            [pltpu.VMEM((CHUNK,), jnp.int32)] * NBUF1,
            [pltpu.VMEM((CHUNK,), jnp.int32)] * NBUF1,
            pltpu.VMEM((NBUF1, CHUNK, H), jnp.float32),
            pltpu.VMEM_SHARED((N, H), jnp.float32),
            [pltpu.SemaphoreType.DMA] * NBUF1,
        ],
    )
    def k(h_hbm, src_hbm, dst_hbm, zero_hbm, out_hbm,
          sidx, didx, sbuf, dbuf, rows, acc, sems):
        c = lax.axis_index("c")
        s = lax.axis_index("s")
        wid = s * NC + c
        cbase = wid * NCHUNK
        pltpu.sync_copy(src_hbm.at[pl.ds(cbase, NCHUNK)], sidx)
        pltpu.sync_copy(dst_hbm.at[pl.ds(cbase, NCHUNK)], didx)

        @pl.when(s == 0)
        def _():
            @pl.when(c == 0)
            def _():
                pltpu.sync_copy(h_hbm, acc)

            @pl.when(c != 0)
            def _():
                pltpu.sync_copy(zero_hbm, acc)

        plsc.subcore_barrier()

        def row_to(buf, src_ref, t):
            for j in range(CHUNK // 16):
                sl = pl.ds(j * 16, 16)
                buf[sl] = src_ref.at[t][sl]

        def fire(t, b):
            row_to(sbuf[b], sidx, t)
            pltpu.async_copy(h_hbm.at[sbuf[b]], rows.at[b], sems[b])

        for b in range(NBUF1):
            fire(b, b)

        def step(t, b):
            pltpu.make_async_copy(h_hbm.at[sbuf[b]], rows.at[b],
                                  sems[b]).wait()
            row_to(dbuf[b], didx, t)
            pltpu.sync_copy(rows.at[b], acc.at[dbuf[b]], add=True)

        def body(jj, carry):
            for b in range(NBUF1):
                t = jj * NBUF1 + b
                step(t, b)
                fire(t + NBUF1, b)
            return carry

        lax.fori_loop(0, (NCHUNK - NBUF1) // NBUF1, body, 0)
        for b in range(NBUF1):
            step(NCHUNK - NBUF1 + b, b)

        plsc.subcore_barrier()
        rpt = N // NS
        pltpu.sync_copy(acc.at[pl.ds(s * rpt, rpt)],
                        out_hbm.at[c].at[pl.ds(s * rpt, rpt)])

    return k(h, src2d, dst2d, zeros_acc)


def _featm_body(h_ref, batch_ref, wq_ref, bq_ref, wk_ref, bk_ref, wv_ref,
                bv_ref, wo_ref, qm_ref, km_ref, vom_ref):
    h = h_ref[...]
    b = batch_ref[...]
    gsz = jnp.zeros((N, 1), jnp.float32)
    for g in range(NGRAPH):
        m = (b == g).astype(jnp.float32)
        gsz = gsz + m * jnp.sum(m)
    sf1 = jnp.log1p(gsz)

    def proj(w_ref_, b_ref_):
        w = w_ref_[...]
        return (jnp.dot(h, w[:H, :], preferred_element_type=jnp.float32)
                + sf1 * w[H:H + 1, :] + b_ref_[...])

    qm_ref[...] = proj(wq_ref, bq_ref)
    km_ref[...] = proj(wk_ref, bk_ref)
    vom_ref[...] = jnp.dot(proj(wv_ref, bv_ref), wo_ref[...],
                           preferred_element_type=jnp.float32)


def _featm(h, batch2d, Wq, bq, Wk, bk, Wv, bv, Wo):
    f32 = jnp.float32
    return pl.pallas_call(
        _featm_body,
        out_shape=(
            jax.ShapeDtypeStruct((N, H), f32),
            jax.ShapeDtypeStruct((N, H), f32),
            jax.ShapeDtypeStruct((N, NEXP), f32),
        ),
    )(h, batch2d, Wq, bq, Wk, bk, Wv, bv, Wo)


def _featd_body(qm_ref, km_ref, vom_ref, pdeg_ref, wq_ref, wk_ref, wv_ref,
                wo_ref, q_ref, k_ref, vo_ref):
    deg = pdeg_ref[0][:, 0:1] + pdeg_ref[1][:, 0:1]
    sf2 = jnp.log1p(deg)
    scale = 1.0 / jnp.sqrt(jnp.float32(ZDIM))
    q_ref[...] = (qm_ref[...] + sf2 * wq_ref[H + 1:H + 2, :]) * scale
    k_ref[...] = km_ref[...] + sf2 * wk_ref[H + 1:H + 2, :]
    wvo = jnp.dot(wv_ref[H + 1:H + 2, :], wo_ref[...],
                  preferred_element_type=jnp.float32)
    vo_ref[...] = vom_ref[...] + sf2 * wvo


def _featd(qm, km, vom, pdeg, Wq, Wk, Wv, Wo):
    f32 = jnp.float32
    return pl.pallas_call(
        _featd_body,
        out_shape=(
            jax.ShapeDtypeStruct((N, H), f32),
            jax.ShapeDtypeStruct((N, H), f32),
            jax.ShapeDtypeStruct((N, NEXP), f32),
        ),
    )(qm, km, vom, pdeg, Wq, Wk, Wv, Wo)


def _attn_body(q_ref, k_ref, vo_ref, bo_ref, epk_ref, w1_ref, w2_ref):
    s = lax.dot_general(q_ref[...], k_ref[...], (((1,), (1,)), ((), ())),
                        preferred_element_type=jnp.float32)
    p = jnp.exp(s)
    l = jnp.sum(p, axis=1, keepdims=True)
    logits = (jnp.dot(p, vo_ref[...], preferred_element_type=jnp.float32) / l
              + bo_ref[...])
    lm = jnp.max(logits, axis=1, keepdims=True)
    le = jnp.exp(logits - lm)
    probs = le / jnp.sum(le, axis=1, keepdims=True)

    v1 = jnp.full((BR, 1), -1.0, jnp.float32)
    i1 = jnp.zeros((BR, 1), jnp.int32)
    for e in range(NEXP):
        ce = probs[:, e:e + 1]
        better = ce > v1
        v1 = jnp.where(better, ce, v1)
        i1 = jnp.where(better, e, i1)
    v2 = jnp.full((BR, 1), -1.0, jnp.float32)
    i2 = jnp.zeros((BR, 1), jnp.int32)
    for e in range(NEXP):
        ce = probs[:, e:e + 1]
        better = (ce > v2) & (i1 != e)
        v2 = jnp.where(better, ce, v2)
        i2 = jnp.where(better, e, i2)
    den = v1 + v2 + 1e-9
    epk_ref[...] = i1 + i2 * NEXP
    w1_ref[...] = v1 / den
    w2_ref[...] = v2 / den


def _attention(q, k, vo, bo):
    f32 = jnp.float32
    i32 = jnp.int32
    nb = N // BR
    return pl.pallas_call(
        _attn_body,
        grid=(nb,),
        in_specs=[
            pl.BlockSpec((BR, H), lambda i: (i, 0)),
            pl.BlockSpec((N, H), lambda i: (0, 0)),
            pl.BlockSpec((N, NEXP), lambda i: (0, 0)),
            pl.BlockSpec((1, NEXP), lambda i: (0, 0)),
        ],
        out_specs=(
            pl.BlockSpec((BR, 1), lambda i: (i, 0)),
            pl.BlockSpec((BR, 1), lambda i: (i, 0)),
            pl.BlockSpec((BR, 1), lambda i: (i, 0)),
        ),
        out_shape=(
            jax.ShapeDtypeStruct((N, 1), i32),
            jax.ShapeDtypeStruct((N, 1), f32),
            jax.ShapeDtypeStruct((N, 1), f32),
        ),
    )(q, k, vo, bo)


def _exp_body(parts_ref, we1_ref, be1_ref, we2_ref, u_ref):
    t = parts_ref[0] + parts_ref[1]
    he = jnp.maximum(
        jnp.dot(t, we1_ref[0], preferred_element_type=jnp.float32)
        + be1_ref[0], 0.0)
    u = jnp.dot(he, we2_ref[0], preferred_element_type=jnp.float32)
    u_ref[...] = u[None]


def _experts(parts, We1, be1, We2):
    return pl.pallas_call(
        _exp_body,
        grid=(NEXP,),
        in_specs=[
            pl.BlockSpec((NC, N, H), lambda e: (0, 0, 0)),
            pl.BlockSpec((1, H, H), lambda e: (e, 0, 0)),
            pl.BlockSpec((1, 1, H), lambda e: (e, 0, 0)),
            pl.BlockSpec((1, H, OUT), lambda e: (e, 0, 0)),
        ],
        out_specs=pl.BlockSpec((1, N, OUT), lambda e: (e, 0, 0)),
        out_shape=jax.ShapeDtypeStruct((NEXP, N, OUT), jnp.float32),
    )(parts, We1, be1.reshape(NEXP, 1, H), We2)


EPT2 = E // NS
NCHUNK2 = EPT2 // CHUNK
NBUF2 = 4


def _seg_gated(uflat, src2d, dst2d, epk, zeros_nh):
    mesh = plsc.VectorSubcoreMesh(core_axis_name="c", subcore_axis_name="s")

    @functools.partial(
        pl.kernel,
        out_type=jax.ShapeDtypeStruct((NC, N, OUT), jnp.float32),
        mesh=mesh,
        scratch_types=[
            pltpu.VMEM((N,), jnp.int32),
            pltpu.VMEM((NCHUNK2, CHUNK), jnp.int32),
            pltpu.VMEM((NCHUNK2, CHUNK), jnp.int32),
            [pltpu.VMEM((CHUNK,), jnp.int32)] * NBUF2,
            [pltpu.VMEM((CHUNK,), jnp.int32)] * NBUF2,
            pltpu.VMEM((NBUF2, CHUNK, OUT), jnp.float32),
            pltpu.VMEM_SHARED((N, OUT), jnp.float32),
            [pltpu.SemaphoreType.DMA] * NBUF2,
        ],
        compiler_params=pltpu.CompilerParams(needs_layout_passes=False),
    )
    def k(u_hbm, src_hbm, dst_hbm, epk_hbm, zero_hbm, out_hbm,
          epkv, sidx, didx, gb, dbuf, rows, acc, sems):
        c = lax.axis_index("c")
        s = lax.axis_index("s")
        cbase = s * NCHUNK2
        pltpu.sync_copy(src_hbm.at[pl.ds(cbase, NCHUNK2)], sidx)
        pltpu.sync_copy(dst_hbm.at[pl.ds(cbase, NCHUNK2)], didx)
        pltpu.sync_copy(epk_hbm, epkv)

        @pl.when(s == 0)
        def _():
            pltpu.sync_copy(zero_hbm, acc)

        plsc.subcore_barrier()

        def fire(t, b):
            for j in range(CHUNK // 16):
                sl = pl.ds(j * 16, 16)
                sv = sidx.at[t][sl]
                dv = didx.at[t][sl]
                ev = plsc.load_gather(epkv, [dv])
                es = jnp.where(c == 0, jnp.bitwise_and(ev, NEXP - 1),
                               jnp.right_shift(ev, 3))
                gb[b][sl] = es * N + sv
            pltpu.async_copy(u_hbm.at[gb[b]], rows.at[b], sems[b])

        def step(t, b):
            pltpu.make_async_copy(u_hbm.at[gb[b]], rows.at[b],
                                  sems[b]).wait()
            for j in range(CHUNK // 16):
                sl = pl.ds(j * 16, 16)
                dbuf[b][sl] = didx.at[t][sl]
            pltpu.sync_copy(rows.at[b], acc.at[dbuf[b]], add=True)

        for b in range(NBUF2):
            fire(b, b)

        def body(jj, carry):
            for b in range(NBUF2):
                t = jj * NBUF2 + b
                step(t, b)
                fire(t + NBUF2, b)
            return carry

        lax.fori_loop(0, (NCHUNK2 - NBUF2) // NBUF2, body, 0)
        for b in range(NBUF2):
            step(NCHUNK2 - NBUF2 + b, b)

        plsc.subcore_barrier()
        rpt = N // NS
        pltpu.sync_copy(acc.at[pl.ds(s * rpt, rpt)],
                        out_hbm.at[c].at[pl.ds(s * rpt, rpt)])

    return k(uflat, src2d, dst2d, epk, zeros_nh)


def _comb_body(u_ref, epk_ref, w1_ref, w2_ref, m_ref, be2_ref, out_ref):
    u = u_ref[...]
    epk = epk_ref[...]
    e1 = jnp.bitwise_and(epk, NEXP - 1)
    e2 = jnp.right_shift(epk, 3)
    sel1 = jnp.zeros((BC, OUT), jnp.float32)
    sel2 = jnp.zeros((BC, OUT), jnp.float32)
    be2 = be2_ref[...]
    for e in range(NEXP):
        ue = u[e] + be2[e:e + 1, :]
        sel1 = sel1 + (e1 == e).astype(jnp.float32) * ue
        sel2 = sel2 + (e2 == e).astype(jnp.float32) * ue
    out_ref[...] = (w1_ref[...] * (sel1 + m_ref[0])
                    + w2_ref[...] * (sel2 + m_ref[1]))


def _combine(U, epk, w1, w2, M, be2):
    nb = N // BC
    return pl.pallas_call(
        _comb_body,
        grid=(nb,),
        in_specs=[
            pl.BlockSpec((NEXP, BC, OUT), lambda i: (0, i, 0)),
            pl.BlockSpec((BC, 1), lambda i: (i, 0)),
            pl.BlockSpec((BC, 1), lambda i: (i, 0)),
            pl.BlockSpec((BC, 1), lambda i: (i, 0)),
            pl.BlockSpec((NC, BC, OUT), lambda i: (0, i, 0)),
            pl.BlockSpec((NEXP, OUT), lambda i: (0, 0)),
        ],
        out_specs=pl.BlockSpec((BC, OUT), lambda i: (i, 0)),
        out_shape=jax.ShapeDtypeStruct((N, OUT), jnp.float32),
    )(U, epk, w1, w2, M, be2)


def kernel(x, edge_index, batch, W_enc, b_enc, Wq, bq, Wk, bk, Wv, bv, Wo, bo,
           We1, be1, We2, be2):
    f32 = jnp.float32
    src2d = edge_index[0].reshape(E // CHUNK, CHUNK)
    dst2d = edge_index[1].reshape(E // CHUNK, CHUNK)

    zeros_nh0 = jnp.zeros((N, H), f32)
    ones_rows = jnp.ones((CHUNK, H), f32)
    pdeg = _deg_sc(dst2d, zeros_nh0, ones_rows)
    h = _encode(x, W_enc, b_enc.reshape(1, H))
    parts = _seg_h(h, src2d, dst2d, zeros_nh0)

    qm, km, vom = _featm(h, batch.reshape(N, 1), Wq, bq.reshape(1, H),
                         Wk, bk.reshape(1, H), Wv, bv.reshape(1, H), Wo)
    q, k, vo = _featd(qm, km, vom, pdeg, Wq, Wk, Wv, Wo)
    epk, w1, w2 = _attention(q, k, vo, bo.reshape(1, NEXP))

    U = _experts(parts, We1, be1, We2)

    M = _seg_gated(U.reshape(N * NEXP, OUT), src2d, dst2d,
                   epk.reshape(N), zeros_nh0)

    return _combine(U, epk, w1, w2, M, be2)


# speedup vs baseline: 1.1613x; 1.0127x over previous
import functools

import jax
import jax.numpy as jnp
from jax import lax
from jax.experimental import pallas as pl
from jax.experimental.pallas import tpu as pltpu
from jax.experimental.pallas import tpu_sc as plsc

N = 4096
E = 65536
H = 128
OUT = 128
NEXP = 8
NGRAPH = 8
ZDIM = 130

NC = 2
NS = 16
NW = NC * NS
EPT = E // NW
CHUNK = 128
NCHUNK = EPT // CHUNK
NBUF1 = 4
NBUF2 = 2
CHUNK2 = 128
NCHUNK2 = EPT // CHUNK2

BR = 512
BC = 1024
EB = 8192


def _enc_body(x_ref, w_ref, b_ref, out_ref):
    xs = x_ref[:, 4:10]
    out_ref[...] = jnp.maximum(
        jnp.dot(xs, w_ref[...], preferred_element_type=jnp.float32)
        + b_ref[...], 0.0)


def _encode(xs, W_enc, b_enc):
    return pl.pallas_call(
        _enc_body,
        out_shape=jax.ShapeDtypeStruct((N, H), jnp.float32),
    )(xs, W_enc, b_enc)


def _deg_sc(dst2d):
    mesh = plsc.VectorSubcoreMesh(core_axis_name="c", subcore_axis_name="s")

    @functools.partial(
        pl.kernel,
        out_type=jax.ShapeDtypeStruct((NW, N // H, H), jnp.float32),
        mesh=mesh,
        scratch_types=[
            pltpu.VMEM((NCHUNK, CHUNK), jnp.int32),
            pltpu.VMEM((N // H, H), jnp.float32),
        ],
        compiler_params=pltpu.CompilerParams(needs_layout_passes=False),
    )
    def k(dst_hbm, out_hbm, didx, cnt):
        c = lax.axis_index("c")
        s = lax.axis_index("s")
        wid = s * NC + c
        cbase = wid * NCHUNK
        pltpu.sync_copy(dst_hbm.at[pl.ds(cbase, NCHUNK)], didx)
        zero = jnp.zeros((16,), jnp.float32)

        def zbody(i, carry):
            for j in range(H // 16):
                cnt[i, pl.ds(j * 16, 16)] = zero
            return carry

        lax.fori_loop(0, N // H, zbody, 0)
        ones = jnp.ones((16,), jnp.float32)

        def body(t, carry):
            for j in range(CHUNK // 16):
                dv = didx.at[t][pl.ds(j * 16, 16)]
                plsc.addupdate_scatter(cnt, [jnp.right_shift(dv, 7),
                                             jnp.bitwise_and(dv, 127)], ones)
            return carry

        lax.fori_loop(0, NCHUNK, body, 0)
        pltpu.sync_copy(cnt, out_hbm.at[wid])

    return k(dst2d)


def _seg_h(h, src2d, dst2d, zeros_acc):
    mesh = plsc.VectorSubcoreMesh(core_axis_name="c", subcore_axis_name="s")

    @functools.partial(
        pl.kernel,
        out_type=jax.ShapeDtypeStruct((NC, N, H), jnp.float32),
        mesh=mesh,
        scratch_types=[
            pltpu.VMEM((NCHUNK, CHUNK), jnp.int32),
            pltpu.VMEM((NCHUNK, CHUNK), jnp.int32),
            [pltpu.VMEM((CHUNK,), jnp.int32)] * NBUF1,
            [pltpu.VMEM((CHUNK,), jnp.int32)] * NBUF1,
            pltpu.VMEM((NBUF1, CHUNK, H), jnp.float32),
            pltpu.VMEM_SHARED((N, H), jnp.float32),
            [pltpu.SemaphoreType.DMA] * NBUF1,
        ],
    )
    def k(h_hbm, src_hbm, dst_hbm, zero_hbm, out_hbm,
          sidx, didx, sbuf, dbuf, rows, acc, sems):
        c = lax.axis_index("c")
        s = lax.axis_index("s")
        wid = s * NC + c
        cbase = wid * NCHUNK
        pltpu.sync_copy(src_hbm.at[pl.ds(cbase, NCHUNK)], sidx)
        pltpu.sync_copy(dst_hbm.at[pl.ds(cbase, NCHUNK)], didx)

        @pl.when(s == 0)
        def _():
            @pl.when(c == 0)
            def _():
                pltpu.sync_copy(h_hbm, acc)

            @pl.when(c != 0)
            def _():
                pltpu.sync_copy(zero_hbm, acc)

        plsc.subcore_barrier()

        def row_to(buf, src_ref, t):
            for j in range(CHUNK // 16):
                sl = pl.ds(j * 16, 16)
                buf[sl] = src_ref.at[t][sl]

        def fire(t, b):
            row_to(sbuf[b], sidx, t)
            pltpu.async_copy(h_hbm.at[sbuf[b]], rows.at[b], sems[b])

        for b in range(NBUF1):
            fire(b, b)

        def step(t, b):
            pltpu.make_async_copy(h_hbm.at[sbuf[b]], rows.at[b],
                                  sems[b]).wait()
            row_to(dbuf[b], didx, t)
            pltpu.sync_copy(rows.at[b], acc.at[dbuf[b]], add=True)

        def body(jj, carry):
            for b in range(NBUF1):
                t = jj * NBUF1 + b
                step(t, b)
                fire(t + NBUF1, b)
            return carry

        lax.fori_loop(0, (NCHUNK - NBUF1) // NBUF1, body, 0)
        for b in range(NBUF1):
            step(NCHUNK - NBUF1 + b, b)

        plsc.subcore_barrier()
        rpt = N // NS
        pltpu.sync_copy(acc.at[pl.ds(s * rpt, rpt)],
                        out_hbm.at[c].at[pl.ds(s * rpt, rpt)])

    return k(h, src2d, dst2d, zeros_acc)


def _featm_body(h_ref, batch_ref, wq_ref, bq_ref, wk_ref, bk_ref, wv_ref,
                bv_ref, wo_ref, qm_ref, km_ref, vom_ref):
    h = h_ref[...]
    b = batch_ref[...]
    gsz = jnp.zeros((N, 1), jnp.float32)
    for g in range(NGRAPH):
        m = (b == g).astype(jnp.float32)
        gsz = gsz + m * jnp.sum(m)
    sf1 = jnp.log1p(gsz)

    def proj(w_ref_, b_ref_):
        w = w_ref_[...]
        return (jnp.dot(h, w[:H, :], preferred_element_type=jnp.float32)
                + sf1 * w[H:H + 1, :] + b_ref_[...])

    qm_ref[...] = proj(wq_ref, bq_ref)
    km_ref[...] = proj(wk_ref, bk_ref)
    vom_ref[...] = jnp.dot(proj(wv_ref, bv_ref), wo_ref[...],
                           preferred_element_type=jnp.float32)


def _featm(h, batch2d, Wq, bq, Wk, bk, Wv, bv, Wo):
    f32 = jnp.float32
    return pl.pallas_call(
        _featm_body,
        out_shape=(
            jax.ShapeDtypeStruct((N, H), f32),
            jax.ShapeDtypeStruct((N, H), f32),
            jax.ShapeDtypeStruct((N, NEXP), f32),
        ),
    )(h, batch2d, Wq, bq, Wk, bk, Wv, bv, Wo)


def _featd_body(qm_ref, km_ref, vom_ref, pdeg_ref, wq_ref, wk_ref, wv_ref,
                wo_ref, q_ref, k_ref, vo_ref):
    dm = jnp.sum(pdeg_ref[...], axis=0)
    rowid = lax.broadcasted_iota(jnp.int32, (N, 1), 0)
    ohi = (jnp.right_shift(rowid, 7)
           == lax.broadcasted_iota(jnp.int32, (1, N // H), 1)
           ).astype(jnp.float32)
    ohl = (jnp.bitwise_and(rowid, 127)
           == lax.broadcasted_iota(jnp.int32, (1, H), 1)).astype(jnp.float32)
    drows = jnp.dot(ohi, dm, preferred_element_type=jnp.float32)
    deg = jnp.sum(drows * ohl, axis=1, keepdims=True)
    sf2 = jnp.log1p(deg)
    scale = 1.0 / jnp.sqrt(jnp.float32(ZDIM))
    q_ref[...] = (qm_ref[...] + sf2 * wq_ref[H + 1:H + 2, :]) * scale
    k_ref[...] = km_ref[...] + sf2 * wk_ref[H + 1:H + 2, :]
    wvo = jnp.dot(wv_ref[H + 1:H + 2, :], wo_ref[...],
                  preferred_element_type=jnp.float32)
    vo_ref[...] = vom_ref[...] + sf2 * wvo


def _featd(qm, km, vom, pdeg, Wq, Wk, Wv, Wo):
    f32 = jnp.float32
    return pl.pallas_call(
        _featd_body,
        out_shape=(
            jax.ShapeDtypeStruct((N, H), f32),
            jax.ShapeDtypeStruct((N, H), f32),
            jax.ShapeDtypeStruct((N, NEXP), f32),
        ),
    )(qm, km, vom, pdeg, Wq, Wk, Wv, Wo)


def _attn_body(q_ref, k_ref, vo_ref, bo_ref, epk_ref, w1_ref, w2_ref):
    s = lax.dot_general(q_ref[...], k_ref[...], (((1,), (1,)), ((), ())),
                        preferred_element_type=jnp.float32)
    p = jnp.exp(s)
    l = jnp.sum(p, axis=1, keepdims=True)
    logits = (jnp.dot(p, vo_ref[...], preferred_element_type=jnp.float32) / l
              + bo_ref[...])
    lm = jnp.max(logits, axis=1, keepdims=True)
    le = jnp.exp(logits - lm)
    probs = le / jnp.sum(le, axis=1, keepdims=True)

    v1 = jnp.full((BR, 1), -1.0, jnp.float32)
    i1 = jnp.zeros((BR, 1), jnp.int32)
    for e in range(NEXP):
        ce = probs[:, e:e + 1]
        better = ce > v1
        v1 = jnp.where(better, ce, v1)
        i1 = jnp.where(better, e, i1)
    v2 = jnp.full((BR, 1), -1.0, jnp.float32)
    i2 = jnp.zeros((BR, 1), jnp.int32)
    for e in range(NEXP):
        ce = probs[:, e:e + 1]
        better = (ce > v2) & (i1 != e)
        v2 = jnp.where(better, ce, v2)
        i2 = jnp.where(better, e, i2)
    den = v1 + v2 + 1e-9
    epk_ref[...] = i1 + i2 * NEXP
    w1_ref[...] = v1 / den
    w2_ref[...] = v2 / den


def _attention(q, k, vo, bo):
    f32 = jnp.float32
    i32 = jnp.int32
    nb = N // BR
    return pl.pallas_call(
        _attn_body,
        grid=(nb,),
        in_specs=[
            pl.BlockSpec((BR, H), lambda i: (i, 0)),
            pl.BlockSpec((N, H), lambda i: (0, 0)),
            pl.BlockSpec((N, NEXP), lambda i: (0, 0)),
            pl.BlockSpec((1, NEXP), lambda i: (0, 0)),
        ],
        out_specs=(
            pl.BlockSpec((BR, 1), lambda i: (i, 0)),
            pl.BlockSpec((BR, 1), lambda i: (i, 0)),
            pl.BlockSpec((BR, 1), lambda i: (i, 0)),
        ),
        out_shape=(
            jax.ShapeDtypeStruct((N, 1), i32),
            jax.ShapeDtypeStruct((N, 1), f32),
            jax.ShapeDtypeStruct((N, 1), f32),
        ),
    )(q, k, vo, bo)


def _exp_body(parts_ref, we1_ref, be1_ref, we2_ref, u_ref):
    t = parts_ref[0] + parts_ref[1]
    he = jnp.maximum(
        jnp.dot(t, we1_ref[0], preferred_element_type=jnp.float32)
        + be1_ref[0], 0.0)
    u = jnp.dot(he, we2_ref[0], preferred_element_type=jnp.float32)
    u_ref[...] = u[None]


def _experts(parts, We1, be1, We2):
    return pl.pallas_call(
        _exp_body,
        grid=(NEXP,),
        in_specs=[
            pl.BlockSpec((NC, N, H), lambda e: (0, 0, 0)),
            pl.BlockSpec((1, H, H), lambda e: (e, 0, 0)),
            pl.BlockSpec((1, 1, H), lambda e: (e, 0, 0)),
            pl.BlockSpec((1, H, OUT), lambda e: (e, 0, 0)),
        ],
        out_specs=pl.BlockSpec((1, N, OUT), lambda e: (e, 0, 0)),
        out_shape=jax.ShapeDtypeStruct((NEXP, N, OUT), jnp.float32),
    )(parts, We1, be1.reshape(NEXP, 1, H), We2)


EPT2 = E // NS
NCHUNK2 = EPT2 // CHUNK
NBUF2 = 4


def _seg_gated(uflat, src2d, dst2d, epk, zeros_nh):
    mesh = plsc.VectorSubcoreMesh(core_axis_name="c", subcore_axis_name="s")

    @functools.partial(
        pl.kernel,
        out_type=jax.ShapeDtypeStruct((NC, N, OUT), jnp.float32),
        mesh=mesh,
        scratch_types=[
            pltpu.VMEM((N,), jnp.int32),
            pltpu.VMEM((NCHUNK2, CHUNK), jnp.int32),
            pltpu.VMEM((NCHUNK2, CHUNK), jnp.int32),
            [pltpu.VMEM((CHUNK,), jnp.int32)] * NBUF2,
            [pltpu.VMEM((CHUNK,), jnp.int32)] * NBUF2,
            pltpu.VMEM((NBUF2, CHUNK, OUT), jnp.float32),
            pltpu.VMEM_SHARED((N, OUT), jnp.float32),
            [pltpu.SemaphoreType.DMA] * NBUF2,
        ],
        compiler_params=pltpu.CompilerParams(needs_layout_passes=False),
    )
    def k(u_hbm, src_hbm, dst_hbm, epk_hbm, zero_hbm, out_hbm,
          epkv, sidx, didx, gb, dbuf, rows, acc, sems):
        c = lax.axis_index("c")
        s = lax.axis_index("s")
        cbase = s * NCHUNK2
        pltpu.sync_copy(src_hbm.at[pl.ds(cbase, NCHUNK2)], sidx)
        pltpu.sync_copy(dst_hbm.at[pl.ds(cbase, NCHUNK2)], didx)
        pltpu.sync_copy(epk_hbm, epkv)

        @pl.when(s == 0)
        def _():
            pltpu.sync_copy(zero_hbm, acc)

        plsc.subcore_barrier()

        def fire(t, b):
            for j in range(CHUNK // 16):
                sl = pl.ds(j * 16, 16)
                sv = sidx.at[t][sl]
                dv = didx.at[t][sl]
                ev = plsc.load_gather(epkv, [dv])
                es = jnp.where(c == 0, jnp.bitwise_and(ev, NEXP - 1),
                               jnp.right_shift(ev, 3))
                gb[b][sl] = es * N + sv
            pltpu.async_copy(u_hbm.at[gb[b]], rows.at[b], sems[b])

        def step(t, b):
            pltpu.make_async_copy(u_hbm.at[gb[b]], rows.at[b],
                                  sems[b]).wait()
            for j in range(CHUNK // 16):
                sl = pl.ds(j * 16, 16)
                dbuf[b][sl] = didx.at[t][sl]
            pltpu.sync_copy(rows.at[b], acc.at[dbuf[b]], add=True)

        for b in range(NBUF2):
            fire(b, b)

        def body(jj, carry):
            for b in range(NBUF2):
                t = jj * NBUF2 + b
                step(t, b)
                fire(t + NBUF2, b)
            return carry

        lax.fori_loop(0, (NCHUNK2 - NBUF2) // NBUF2, body, 0)
        for b in range(NBUF2):
            step(NCHUNK2 - NBUF2 + b, b)

        plsc.subcore_barrier()
        rpt = N // NS
        pltpu.sync_copy(acc.at[pl.ds(s * rpt, rpt)],
                        out_hbm.at[c].at[pl.ds(s * rpt, rpt)])

    return k(uflat, src2d, dst2d, epk, zeros_nh)


def _comb_body(u_ref, epk_ref, w1_ref, w2_ref, m_ref, be2_ref, out_ref):
    u = u_ref[...]
    epk = epk_ref[...]
    e1 = jnp.bitwise_and(epk, NEXP - 1)
    e2 = jnp.right_shift(epk, 3)
    sel1 = jnp.zeros((BC, OUT), jnp.float32)
    sel2 = jnp.zeros((BC, OUT), jnp.float32)
    be2 = be2_ref[...]
    for e in range(NEXP):
        ue = u[e] + be2[e:e + 1, :]
        sel1 = sel1 + (e1 == e).astype(jnp.float32) * ue
        sel2 = sel2 + (e2 == e).astype(jnp.float32) * ue
    out_ref[...] = (w1_ref[...] * (sel1 + m_ref[0])
                    + w2_ref[...] * (sel2 + m_ref[1]))


def _combine(U, epk, w1, w2, M, be2):
    nb = N // BC
    return pl.pallas_call(
        _comb_body,
        grid=(nb,),
        in_specs=[
            pl.BlockSpec((NEXP, BC, OUT), lambda i: (0, i, 0)),
            pl.BlockSpec((BC, 1), lambda i: (i, 0)),
            pl.BlockSpec((BC, 1), lambda i: (i, 0)),
            pl.BlockSpec((BC, 1), lambda i: (i, 0)),
            pl.BlockSpec((NC, BC, OUT), lambda i: (0, i, 0)),
            pl.BlockSpec((NEXP, OUT), lambda i: (0, 0)),
        ],
        out_specs=pl.BlockSpec((BC, OUT), lambda i: (i, 0)),
        out_shape=jax.ShapeDtypeStruct((N, OUT), jnp.float32),
    )(U, epk, w1, w2, M, be2)


def kernel(x, edge_index, batch, W_enc, b_enc, Wq, bq, Wk, bk, Wv, bv, Wo, bo,
           We1, be1, We2, be2):
    f32 = jnp.float32
    src2d = edge_index[0].reshape(E // CHUNK, CHUNK)
    dst2d = edge_index[1].reshape(E // CHUNK, CHUNK)

    zeros_nh0 = jnp.zeros((N, H), f32)
    pdeg = _deg_sc(dst2d)
    h = _encode(x, W_enc, b_enc.reshape(1, H))
    parts = _seg_h(h, src2d, dst2d, zeros_nh0)

    qm, km, vom = _featm(h, batch.reshape(N, 1), Wq, bq.reshape(1, H),
                         Wk, bk.reshape(1, H), Wv, bv.reshape(1, H), Wo)
    q, k, vo = _featd(qm, km, vom, pdeg, Wq, Wk, Wv, Wo)
    epk, w1, w2 = _attention(q, k, vo, bo.reshape(1, NEXP))

    U = _experts(parts, We1, be1, We2)

    M = _seg_gated(U.reshape(N * NEXP, OUT), src2d, dst2d,
                   epk.reshape(N), zeros_nh0)

    return _combine(U, epk, w1, w2, M, be2)
